# Initial kernel scaffold; baseline (speedup 1.0000x reference)
import jax
import jax.numpy as jnp
from jax.experimental import pallas as pl


def kernel(pos, faces, input_feature, Ws1, Wn1, b1, Wp1, Wf1, bf1, Ws2, Wn2, b2, Wp2, Wf2, bf2, Ws3, Wn3, b3, Wp3):
    raise NotImplementedError("write your pallas kernel here")


# speedup vs baseline: 7.1423x; 7.1423x over previous
import functools

import jax
import jax.numpy as jnp
from jax import lax
from jax.experimental import pallas as pl
from jax.experimental.pallas import tpu as pltpu
from jax.experimental.pallas import tpu_sc as plsc

BM = 400

_f32 = jnp.float32


def _mesh():
    return plsc.VectorSubcoreMesh(core_axis_name="c", subcore_axis_name="s")


_SC_PARAMS = pltpu.CompilerParams(use_tc_tiling_on_sc=False)


def _gather(table, idx, W, Mpad):
    Mt = Mpad // 32
    n_g = Mt // 128

    @functools.partial(
        pl.kernel,
        out_type=jax.ShapeDtypeStruct((Mpad, W), _f32),
        mesh=_mesh(),
        compiler_params=_SC_PARAMS,
        scratch_types=[
            pltpu.VMEM((128,), jnp.int32),
            pltpu.VMEM((128, W), _f32),
            pltpu.SemaphoreType.DMA,
        ],
    )
    def k(table_hbm, idx_hbm, out_hbm, idx_v, rows_v, sem):
        c = lax.axis_index("c")
        s = lax.axis_index("s")
        wid = s * 2 + c
        base = wid * Mt

        def body(i, _):
            off = base + i * 128
            pltpu.sync_copy(idx_hbm.at[pl.ds(off, 128)], idx_v)
            pltpu.async_copy(table_hbm.at[idx_v], rows_v, sem).wait()
            pltpu.sync_copy(rows_v, out_hbm.at[pl.ds(off, 128)])
            return _

        lax.fori_loop(0, n_g, body, None)

    return k(table, idx)


def _scatter_kernel(vals, idx3, zblk, *, F, N_acc, W, n_slices, s_per, mode):
    Ft = F // 16
    Kt = (Ft + 127) // 128
    n_full = Ft // 128
    rem = Ft - n_full * 128
    Nz = N_acc // 16
    n_zf = Nz // 1024
    zrem = Nz - n_zf * 1024
    Ctot = n_slices * W

    @functools.partial(
        pl.kernel,
        out_type=jax.ShapeDtypeStruct((N_acc, Ctot), _f32),
        mesh=_mesh(),
        compiler_params=_SC_PARAMS,
        scratch_types=[
            pltpu.VMEM((3, 128), jnp.int32),
            pltpu.VMEM((128, W), _f32),
            pltpu.VMEM_SHARED((N_acc, W), _f32),
        ],
    )
    def k(vals_hbm, idx_hbm, z_hbm, out_hbm, idx_v, rows_v, acc):
        c = lax.axis_index("c")
        t = lax.axis_index("s")
        base_n = t * Nz
        f_base = t * Ft

        for slice_id in range(n_slices):
            owner = slice_id // s_per

            @pl.when(c == owner)
            def _():
                c0 = slice_id * W
                for z in range(n_zf):
                    pltpu.sync_copy(z_hbm.at[pl.ds(0, 1024), pl.ds(0, W)],
                                    acc.at[pl.ds(base_n + z * 1024, 1024)])
                if zrem:
                    pltpu.sync_copy(z_hbm.at[pl.ds(0, zrem), pl.ds(0, W)],
                                    acc.at[pl.ds(base_n + n_zf * 1024, zrem)])
                plsc.subcore_barrier()

                if mode == "s1":
                    def body(i, _):
                        pltpu.sync_copy(idx_hbm.at[t, i], idx_v)
                        pltpu.sync_copy(
                            vals_hbm.at[pl.ds(f_base + i * 128, 128),
                                        pl.ds(c0, W)], rows_v)
                        for j in range(3):
                            pltpu.sync_copy(rows_v, acc.at[idx_v.at[j]],
                                            add=True)
                        return _
                else:
                    def body(i, _):
                        pltpu.sync_copy(idx_hbm.at[t, i], idx_v)
                        for j in range(3):
                            pltpu.sync_copy(
                                vals_hbm.at[pl.ds(f_base + i * 128, 128),
                                            pl.ds(j * 64 + c0, W)], rows_v)
                            pltpu.sync_copy(rows_v, acc.at[idx_v.at[j]],
                                            add=True)
                        return _

                lax.fori_loop(0, n_full, body, None)
                if rem:
                    pltpu.sync_copy(idx_hbm.at[t, n_full], idx_v)
                    if mode == "s1":
                        pltpu.sync_copy(
                            vals_hbm.at[pl.ds(f_base + n_full * 128, rem),
                                        pl.ds(c0, W)],
                            rows_v.at[pl.ds(0, rem)])
                        for j in range(3):
                            pltpu.sync_copy(rows_v, acc.at[idx_v.at[j]],
                                            add=True)
                    else:
                        for j in range(3):
                            pltpu.sync_copy(
                                vals_hbm.at[pl.ds(f_base + n_full * 128, rem),
                                            pl.ds(j * 64 + c0, W)],
                                rows_v.at[pl.ds(0, rem)])
                            pltpu.sync_copy(rows_v, acc.at[idx_v.at[j]],
                                            add=True)
                plsc.subcore_barrier()
                pltpu.sync_copy(acc.at[pl.ds(base_n, Nz)],
                                out_hbm.at[pl.ds(base_n, Nz), pl.ds(c0, W)])

    return k(vals, idx3, zblk)


def _t0_extract(pg, infeat, F):
    def body(pa, pb, pc, x, o):
        v0, v1, v2 = pa[:, 0:3], pb[:, 0:3], pc[:, 0:3]
        center = (v0 + v1 + v2) * (1.0 / 3.0)
        e1 = v1 - v0
        e2 = v2 - v0
        cr = jnp.stack([
            e1[:, 1] * e2[:, 2] - e1[:, 2] * e2[:, 1],
            e1[:, 2] * e2[:, 0] - e1[:, 0] * e2[:, 2],
            e1[:, 0] * e2[:, 1] - e1[:, 1] * e2[:, 0],
        ], axis=1)
        area = 0.5 * jnp.sqrt(jnp.sum(cr * cr, axis=1, keepdims=True) + 1e-12)
        normal = cr / (2.0 * area + 1e-8)
        ones = jnp.ones((pa.shape[0], 1), _f32)
        zer = jnp.zeros((pa.shape[0], 8), _f32)
        o[...] = jnp.concatenate([center, normal, area, x[...], ones, zer],
                                 axis=1)

    nb = F // BM
    sp8 = lambda j: pl.BlockSpec((BM, 8), lambda i, j=j: (i + j * nb, 0))
    return pl.pallas_call(
        body,
        grid=(nb,),
        in_specs=[sp8(0), sp8(1), sp8(2),
                  pl.BlockSpec((BM, 16), lambda i: (i, 0))],
        out_specs=pl.BlockSpec((BM, 32), lambda i: (i, 0)),
        out_shape=jax.ShapeDtypeStruct((F, 32), _f32),
    )(pg, pg, pg, infeat)


def _t1_nm_matmul(s1, cnt, W, N, C):
    def body(s_ref, c_ref, w_ref, o):
        inv = 1.0 / jnp.maximum(c_ref[...], 1.0)
        o[...] = jnp.dot(s_ref[...] * inv, w_ref[...],
                         preferred_element_type=_f32)

    return pl.pallas_call(
        body,
        grid=(N // BM,),
        in_specs=[pl.BlockSpec((BM, C), lambda i: (i, 0)),
                  pl.BlockSpec((BM, 1), lambda i: (i, 0)),
                  pl.BlockSpec((C, 64), lambda i: (0, 0))],
        out_specs=pl.BlockSpec((BM, 64), lambda i: (i, 0)),
        out_shape=jax.ShapeDtypeStruct((N, 64), _f32),
    )(s1, cnt, W)


def _t2_children(feat, gnm, pg, A, S, b, Wp8, F, C, Wf=None, bf=None):
    with_nf = Wf is not None

    def body(*refs):
        if with_nf:
            (f_ref, g0r, g1r, g2r, pa, pb, pc, a_ref, s_ref, b_ref, wp_ref,
             wf_ref, bf_ref, g3d_o, mp_o, nf_o) = refs
        else:
            (f_ref, g0r, g1r, g2r, pa, pb, pc, a_ref, s_ref, b_ref,
             wp_ref, g3d_o, mp_o) = refs
        ft = f_ref[...]
        bb = b_ref[...]
        P = jnp.dot(ft, a_ref[...], preferred_element_type=_f32) + bb
        Q = jnp.dot(ft, s_ref[...], preferred_element_type=_f32) + bb
        g0 = jnp.maximum(P + g0r[...], 0.0)
        g1 = jnp.maximum(P + g1r[...], 0.0)
        g2 = jnp.maximum(P + g2r[...], 0.0)
        g3 = jnp.maximum(Q, 0.0)
        g3d_o[...] = jnp.concatenate([g0, g1, g2, g3], axis=1)
        third = 1.0 / 3.0
        wp = wp_ref[...]
        m01 = jnp.dot((g0 + g1 + g3) * third, wp, preferred_element_type=_f32)
        m12 = jnp.dot((g1 + g2 + g3) * third, wp, preferred_element_type=_f32)
        m20 = jnp.dot((g0 + g2 + g3) * third, wp, preferred_element_type=_f32)
        p01 = 0.5 * (pa[...] + pb[...]) + m01
        p12 = 0.5 * (pb[...] + pc[...]) + m12
        p20 = 0.5 * (pc[...] + pa[...]) + m20
        mp_o[...] = jnp.concatenate([p01, p12, p20], axis=1)
        if with_nf:
            wf = wf_ref[...]
            bf_ = bf_ref[...]
            nf_o[...] = jnp.concatenate(
                [jnp.dot(g, wf, preferred_element_type=_f32) + bf_
                 for g in (g0, g1, g2, g3)], axis=1)

    nb = F // BM
    sp64 = lambda j: pl.BlockSpec((BM, 64), lambda i, j=j: (i + j * nb, 0))
    sp8 = lambda j: pl.BlockSpec((BM, 8), lambda i, j=j: (i + j * nb, 0))
    full = lambda r, c: pl.BlockSpec((r, c), lambda i: (0, 0))
    in_specs = [pl.BlockSpec((BM, C), lambda i: (i, 0)),
                sp64(0), sp64(1), sp64(2), sp8(0), sp8(1), sp8(2),
                full(C, 64), full(C, 64), full(1, 64), full(64, 8)]
    args = [feat, gnm, gnm, gnm, pg, pg, pg, A, S, b, Wp8]
    out_specs = [pl.BlockSpec((BM, 256), lambda i: (i, 0)),
                 pl.BlockSpec((BM, 24), lambda i: (i, 0))]
    out_shape = [jax.ShapeDtypeStruct((F, 256), _f32),
                 jax.ShapeDtypeStruct((F, 24), _f32)]
    if with_nf:
        in_specs += [full(64, 64), full(1, 64)]
        args += [Wf, bf]
        out_specs.append(pl.BlockSpec((BM, 256), lambda i: (i, 0)))
        out_shape.append(jax.ShapeDtypeStruct((F, 256), _f32))
    return pl.pallas_call(
        body, grid=(nb,), in_specs=in_specs, out_specs=out_specs,
        out_shape=out_shape)(*args)


def _t3_pos(posP, s2, cnt, Wp8, N):
    def body(p_ref, s_ref, c_ref, w_ref, o):
        inv = 1.0 / jnp.maximum(c_ref[...], 1.0)
        o[...] = p_ref[...] + jnp.dot(s_ref[...] * inv, w_ref[...],
                                      preferred_element_type=_f32)

    return pl.pallas_call(
        body,
        grid=(N // BM,),
        in_specs=[pl.BlockSpec((BM, 8), lambda i: (i, 0)),
                  pl.BlockSpec((BM, 64), lambda i: (i, 0)),
                  pl.BlockSpec((BM, 1), lambda i: (i, 0)),
                  pl.BlockSpec((64, 8), lambda i: (0, 0))],
        out_specs=pl.BlockSpec((BM, 8), lambda i: (i, 0)),
        out_shape=jax.ShapeDtypeStruct((N, 8), _f32),
    )(posP, s2, cnt, Wp8)


def _pad_rows(w, rows):
    return jnp.concatenate(
        [w, jnp.zeros((rows - w.shape[0], w.shape[1]), _f32)], axis=0)


def _gather_idx(faces, F):
    flat = jnp.concatenate([faces[:, 0], faces[:, 1], faces[:, 2]])
    Mpad = -(-3 * F // 4096) * 4096
    return jnp.concatenate(
        [flat, jnp.zeros((Mpad - 3 * F,), jnp.int32)]), Mpad


def _scatter_idx(faces, F, N):
    Ft = F // 16
    Kt = (Ft + 127) // 128
    cols = []
    for j in range(3):
        cj = faces[:, j].reshape(16, Ft)
        cj = jnp.pad(cj, ((0, 0), (0, Kt * 128 - Ft)), constant_values=N)
        cols.append(cj.reshape(16, Kt, 128))
    return jnp.stack(cols, axis=2)


def _build_faces(faces, N, F):
    a, b, c = faces[:, 0], faces[:, 1], faces[:, 2]
    i01 = N + jnp.arange(F, dtype=faces.dtype)
    i12 = i01 + F
    i20 = i12 + F
    c0 = jnp.stack([a, i01, i20], axis=1)
    c1 = jnp.stack([i01, b, i12], axis=1)
    c2 = jnp.stack([i20, i12, c], axis=1)
    c3 = jnp.stack([i01, i12, i20], axis=1)
    return jnp.stack([c0, c1, c2, c3], axis=1).reshape(-1, 3)


def _round(posP, faces, feat, cnt, Ws, Wn, b, Wp, Wf, bf, *, N, F, C,
           s1W, s1_sl, s2W, s2_sl, zblk, s1_pre=None, gidx_pre=None,
           sidx_pre=None, pg_pre=None):
    N_acc = -(-(N + 1) // 16) * 16
    if gidx_pre is not None:
        gidx, Mpad = gidx_pre
    else:
        gidx, Mpad = _gather_idx(faces, F)
    sidx = sidx_pre if sidx_pre is not None else _scatter_idx(faces, F, N)
    b2 = b.reshape(1, 64)
    A = Ws + (2.0 / 3.0) * Wn
    S = Ws + Wn
    Wn3 = Wn * (1.0 / 3.0)
    Wp8 = jnp.concatenate([Wp, jnp.zeros((64, 5), _f32)], axis=1)

    if s1_pre is not None:
        s1 = s1_pre
    else:
        s1 = _scatter3(feat, sidx, zblk, F=F, N_acc=N_acc, W=s1W,
                       n_slices=s1_sl, mode="s1")
    nmW3 = _t1_nm_matmul(s1, cnt, Wn3, N, C)
    gnm = _gather(nmW3, gidx, 64, Mpad)
    pg = pg_pre if pg_pre is not None else _gather(posP, gidx, 8, Mpad)
    outs = _t2_children(feat, gnm, pg, A, S, b2, Wp8, F, C, Wf,
                        bf.reshape(1, 64) if bf is not None else None)
    if Wf is not None:
        g3d, mp, nf = outs
        feat_next = nf.reshape(F * 4, 64)
    else:
        g3d, mp = outs
        feat_next = None
    s2 = _scatter3(g3d, sidx, zblk, F=F, N_acc=N_acc, W=s2W,
                   n_slices=s2_sl, mode="s2")
    newposO = _t3_pos(posP, s2, cnt, Wp8, N)
    posP_next = jnp.concatenate(
        [newposO, mp[:, 0:8], mp[:, 8:16], mp[:, 16:24]], axis=0)
    faces_next = _build_faces(faces, N, F)
    cnt_next = jnp.concatenate([cnt, jnp.full((3 * F, 1), 3.0, _f32)], axis=0)
    return posP_next, faces_next, feat_next, cnt_next


def _scatter3(vals, sidx, zblk, *, F, N_acc, W, n_slices, mode):
    s_per = max(1, n_slices // 2)
    return _scatter_kernel(vals, sidx, zblk, F=F, N_acc=N_acc, W=W,
                           n_slices=n_slices, s_per=s_per, mode=mode)


def kernel(pos, faces, input_feature, Ws1, Wn1, b1, Wp1, Wf1, bf1,
           Ws2, Wn2, b2, Wp2, Wf2, bf2, Ws3, Wn3, b3, Wp3):
    N0, F0 = pos.shape[0], faces.shape[0]
    zblk = jnp.zeros((1024, 64), _f32)
    posP0 = jnp.concatenate([pos, jnp.zeros((N0, 5), _f32)], axis=1)

    gidx0, Mpad0 = _gather_idx(faces, F0)
    pg0 = _gather(posP0, gidx0, 8, Mpad0)
    ff = _t0_extract(pg0, input_feature, F0)

    Ws1p = _pad_rows(Ws1, 32)
    Wn1p = _pad_rows(Wn1, 32)

    N_acc0 = -(-(N0 + 1) // 16) * 16
    sidx0 = _scatter_idx(faces, F0, N0)
    s1r1 = _scatter3(ff, sidx0, zblk, F=F0, N_acc=N_acc0, W=16,
                     n_slices=2, mode="s1")
    cnt1 = s1r1[:N0, 23:24]

    p1P, f1, feat1, cnt2 = _round(
        posP0, faces, ff, cnt1, Ws1p, Wn1p, b1, Wp1, Wf1, bf1,
        N=N0, F=F0, C=32, s1W=16, s1_sl=2, s2W=32, s2_sl=2, zblk=zblk,
        s1_pre=s1r1, gidx_pre=(gidx0, Mpad0), sidx_pre=sidx0, pg_pre=pg0,
    )

    N1, F1 = N0 + 3 * F0, 4 * F0
    p2P, f2, feat2, cnt3 = _round(
        p1P, f1, feat1, cnt2, Ws2, Wn2, b2, Wp2, Wf2, bf2,
        N=N1, F=F1, C=64, s1W=32, s1_sl=2, s2W=32, s2_sl=2, zblk=zblk,
    )

    N2, F2 = N1 + 3 * F1, 4 * F1
    p3P, f3, _, _ = _round(
        p2P, f2, feat2, cnt3, Ws3, Wn3, b3, Wp3, None, None,
        N=N2, F=F2, C=64, s1W=16, s1_sl=4, s2W=16, s2_sl=4, zblk=zblk,
    )

    return (p1P[:, 0:3], p2P[:, 0:3], p3P[:, 0:3], f1, f2, f3)


# speedup vs baseline: 7.8143x; 1.0941x over previous
import functools

import jax
import jax.numpy as jnp
from jax import lax
from jax.experimental import pallas as pl
from jax.experimental.pallas import tpu as pltpu
from jax.experimental.pallas import tpu_sc as plsc

BM = 400

_f32 = jnp.float32


def _mesh():
    return plsc.VectorSubcoreMesh(core_axis_name="c", subcore_axis_name="s")


_SC_PARAMS = pltpu.CompilerParams(use_tc_tiling_on_sc=False)


def _gather(table, idx, W, Mpad):
    Mt = Mpad // 32
    n_g = Mt // 128

    @functools.partial(
        pl.kernel,
        out_type=jax.ShapeDtypeStruct((Mpad, W), _f32),
        mesh=_mesh(),
        compiler_params=_SC_PARAMS,
        scratch_types=[
            pltpu.VMEM((2, 128), jnp.int32),
            pltpu.VMEM((2, 128, W), _f32),
            pltpu.SemaphoreType.DMA,
            pltpu.SemaphoreType.DMA,
            pltpu.SemaphoreType.DMA,
            pltpu.SemaphoreType.DMA,
        ],
    )
    def k(table_hbm, idx_hbm, out_hbm, idx_v, rows_v, g0, g1, s0, s1):
        c = lax.axis_index("c")
        s = lax.axis_index("s")
        wid = s * 2 + c
        base = wid * Mt
        gsem = (g0, g1)
        ssem = (s0, s1)

        def chunk(i, b, p):
            off = base + i * 128

            @pl.when(p > 0)
            def _():
                pltpu.make_async_copy(
                    rows_v.at[b], out_hbm.at[pl.ds(0, 128)], ssem[b]).wait()

            pltpu.sync_copy(idx_hbm.at[pl.ds(off, 128)], idx_v.at[b])
            pltpu.async_copy(table_hbm.at[idx_v.at[b]], rows_v.at[b], gsem[b])

        def fin(i, b):
            off = base + i * 128
            pltpu.make_async_copy(
                table_hbm.at[idx_v.at[b]], rows_v.at[b], gsem[b]).wait()
            pltpu.async_copy(rows_v.at[b], out_hbm.at[pl.ds(off, 128)],
                             ssem[b])

        def body(p, _):
            i0 = 2 * p
            chunk(i0, 0, p)
            chunk(i0 + 1, 1, p)
            fin(i0, 0)
            fin(i0 + 1, 1)
            return _

        lax.fori_loop(0, n_g // 2, body, None)
        for b in range(2):
            pltpu.make_async_copy(
                rows_v.at[b], out_hbm.at[pl.ds(0, 128)], ssem[b]).wait()

    return k(table, idx)


def _scatter_kernel(vals, idx3, zblk, *, M, N_acc, W, n_slices, s_per,
                    n_scat):
    Mt = M // 16
    Kt = (Mt + 127) // 128
    n_full = Mt // 128
    rem = Mt - n_full * 128
    Nz = N_acc // 16
    n_zf = Nz // 1024
    zrem = Nz - n_zf * 1024
    Ctot = n_slices * W

    @functools.partial(
        pl.kernel,
        out_type=jax.ShapeDtypeStruct((N_acc, Ctot), _f32),
        mesh=_mesh(),
        compiler_params=_SC_PARAMS,
        scratch_types=[
            pltpu.VMEM((2, n_scat, 128), jnp.int32),
            pltpu.VMEM((2, 128, W), _f32),
            pltpu.VMEM_SHARED((N_acc, W), _f32),
            pltpu.SemaphoreType.DMA,
            pltpu.SemaphoreType.DMA,
        ],
    )
    def k(vals_hbm, idx_hbm, z_hbm, out_hbm, idx_v, rows_v, acc, sm0, sm1):
        c = lax.axis_index("c")
        t = lax.axis_index("s")
        base_n = t * Nz
        f_base = t * Mt
        sems = (sm0, sm1)

        def drain(b):
            for j in range(n_scat):
                pltpu.make_async_copy(rows_v.at[b],
                                      acc.at[idx_v.at[b, j]],
                                      sems[b]).wait()

        for slice_id in range(n_slices):
            owner = slice_id // s_per

            @pl.when(c == owner)
            def _():
                c0 = slice_id * W
                for z in range(n_zf):
                    pltpu.sync_copy(z_hbm.at[pl.ds(0, 1024), pl.ds(0, W)],
                                    acc.at[pl.ds(base_n + z * 1024, 1024)])
                if zrem:
                    pltpu.sync_copy(z_hbm.at[pl.ds(0, zrem), pl.ds(0, W)],
                                    acc.at[pl.ds(base_n + n_zf * 1024, zrem)])
                plsc.subcore_barrier()

                def issue(i, b):
                    pltpu.sync_copy(idx_hbm.at[t, i], idx_v.at[b])
                    pltpu.sync_copy(
                        vals_hbm.at[pl.ds(f_base + i * 128, 128),
                                    pl.ds(c0, W)], rows_v.at[b])
                    for j in range(n_scat):
                        pltpu.async_copy(rows_v.at[b],
                                         acc.at[idx_v.at[b, j]],
                                         sems[b], add=True)

                def pbody(p, _):
                    @pl.when(p > 0)
                    def _():
                        drain(0)
                    issue(2 * p, 0)

                    @pl.when(p > 0)
                    def _():
                        drain(1)
                    issue(2 * p + 1, 1)
                    return _

                n_pairs = n_full // 2
                lax.fori_loop(0, n_pairs, pbody, None)
                if n_full % 2:
                    drain(0)
                    issue(2 * n_pairs, 0)
                if rem:
                    b = n_full % 2
                    drain(b)
                    pltpu.sync_copy(idx_hbm.at[t, n_full], idx_v.at[b])
                    pltpu.sync_copy(
                        vals_hbm.at[pl.ds(f_base + n_full * 128, rem),
                                    pl.ds(c0, W)],
                        rows_v.at[b, pl.ds(0, rem)])
                    for j in range(n_scat):
                        pltpu.async_copy(rows_v.at[b],
                                         acc.at[idx_v.at[b, j]],
                                         sems[b], add=True)
                drain(0)
                drain(1)
                plsc.subcore_barrier()
                pltpu.sync_copy(acc.at[pl.ds(base_n, Nz)],
                                out_hbm.at[pl.ds(base_n, Nz), pl.ds(c0, W)])

    return k(vals, idx3, zblk)


def _t0_extract(pg, infeat, F):
    def body(pa, pb, pc, x, o):
        v0, v1, v2 = pa[:, 0:3], pb[:, 0:3], pc[:, 0:3]
        center = (v0 + v1 + v2) * (1.0 / 3.0)
        e1 = v1 - v0
        e2 = v2 - v0
        cr = jnp.stack([
            e1[:, 1] * e2[:, 2] - e1[:, 2] * e2[:, 1],
            e1[:, 2] * e2[:, 0] - e1[:, 0] * e2[:, 2],
            e1[:, 0] * e2[:, 1] - e1[:, 1] * e2[:, 0],
        ], axis=1)
        area = 0.5 * jnp.sqrt(jnp.sum(cr * cr, axis=1, keepdims=True) + 1e-12)
        normal = cr / (2.0 * area + 1e-8)
        ones = jnp.ones((pa.shape[0], 1), _f32)
        zer = jnp.zeros((pa.shape[0], 8), _f32)
        o[...] = jnp.concatenate([center, normal, area, x[...], ones, zer],
                                 axis=1)

    nb = F // BM
    sp8 = lambda j: pl.BlockSpec((BM, 8), lambda i, j=j: (i + j * nb, 0))
    return pl.pallas_call(
        body,
        grid=(nb,),
        in_specs=[sp8(0), sp8(1), sp8(2),
                  pl.BlockSpec((BM, 16), lambda i: (i, 0))],
        out_specs=pl.BlockSpec((BM, 32), lambda i: (i, 0)),
        out_shape=jax.ShapeDtypeStruct((F, 32), _f32),
    )(pg, pg, pg, infeat)


def _t1_nm_matmul(s1, cnt, W, N, C):
    def body(s_ref, c_ref, w_ref, o):
        inv = 1.0 / jnp.maximum(c_ref[...], 1.0)
        o[...] = jnp.dot(s_ref[...] * inv, w_ref[...],
                         preferred_element_type=_f32)

    return pl.pallas_call(
        body,
        grid=(N // BM,),
        in_specs=[pl.BlockSpec((BM, C), lambda i: (i, 0)),
                  pl.BlockSpec((BM, 1), lambda i: (i, 0)),
                  pl.BlockSpec((C, 64), lambda i: (0, 0))],
        out_specs=pl.BlockSpec((BM, 64), lambda i: (i, 0)),
        out_shape=jax.ShapeDtypeStruct((N, 64), _f32),
    )(s1, cnt, W)


def _t2_children(feat, gnm, pg, A, S, b, Wp8, F, C, Wf=None, bf=None):
    with_nf = Wf is not None

    def body(*refs):
        if with_nf:
            (f_ref, g0r, g1r, g2r, pa, pb, pc, a_ref, s_ref, b_ref, wp_ref,
             wf_ref, bf_ref, g3d_o, mp_o, nf_o) = refs
        else:
            (f_ref, g0r, g1r, g2r, pa, pb, pc, a_ref, s_ref, b_ref,
             wp_ref, g3d_o, mp_o) = refs
        ft = f_ref[...]
        bb = b_ref[...]
        P = jnp.dot(ft, a_ref[...], preferred_element_type=_f32) + bb
        Q = jnp.dot(ft, s_ref[...], preferred_element_type=_f32) + bb
        g0 = jnp.maximum(P + g0r[...], 0.0)
        g1 = jnp.maximum(P + g1r[...], 0.0)
        g2 = jnp.maximum(P + g2r[...], 0.0)
        g3 = jnp.maximum(Q, 0.0)
        g3d_o[...] = jnp.stack([g0, g1, g2], axis=0)
        third = 1.0 / 3.0
        wp = wp_ref[...]
        m01 = jnp.dot((g0 + g1 + g3) * third, wp, preferred_element_type=_f32)
        m12 = jnp.dot((g1 + g2 + g3) * third, wp, preferred_element_type=_f32)
        m20 = jnp.dot((g0 + g2 + g3) * third, wp, preferred_element_type=_f32)
        p01 = 0.5 * (pa[...] + pb[...]) + m01
        p12 = 0.5 * (pb[...] + pc[...]) + m12
        p20 = 0.5 * (pc[...] + pa[...]) + m20
        mp_o[...] = jnp.concatenate([p01, p12, p20], axis=1)
        if with_nf:
            wf = wf_ref[...]
            bf_ = bf_ref[...]
            nf_o[...] = jnp.stack(
                [jnp.dot(g, wf, preferred_element_type=_f32) + bf_
                 for g in (g0, g1, g2, g3)],
                axis=1).reshape(4 * g0.shape[0], 64)

    nb = F // BM
    sp64 = lambda j: pl.BlockSpec((BM, 64), lambda i, j=j: (i + j * nb, 0))
    sp8 = lambda j: pl.BlockSpec((BM, 8), lambda i, j=j: (i + j * nb, 0))
    full = lambda r, c: pl.BlockSpec((r, c), lambda i: (0, 0))
    in_specs = [pl.BlockSpec((BM, C), lambda i: (i, 0)),
                sp64(0), sp64(1), sp64(2), sp8(0), sp8(1), sp8(2),
                full(C, 64), full(C, 64), full(1, 64), full(64, 8)]
    args = [feat, gnm, gnm, gnm, pg, pg, pg, A, S, b, Wp8]
    out_specs = [pl.BlockSpec((3, BM, 64), lambda i: (0, i, 0)),
                 pl.BlockSpec((BM, 24), lambda i: (i, 0))]
    out_shape = [jax.ShapeDtypeStruct((3, F, 64), _f32),
                 jax.ShapeDtypeStruct((F, 24), _f32)]
    if with_nf:
        in_specs += [full(64, 64), full(1, 64)]
        args += [Wf, bf]
        out_specs.append(pl.BlockSpec((4 * BM, 64), lambda i: (i, 0)))
        out_shape.append(jax.ShapeDtypeStruct((4 * F, 64), _f32))
    return pl.pallas_call(
        body, grid=(nb,), in_specs=in_specs, out_specs=out_specs,
        out_shape=out_shape)(*args)


def _t3_pos(posP, s2, cnt, Wp8, N):
    def body(p_ref, s_ref, c_ref, w_ref, o):
        inv = 1.0 / jnp.maximum(c_ref[...], 1.0)
        o[...] = p_ref[...] + jnp.dot(s_ref[...] * inv, w_ref[...],
                                      preferred_element_type=_f32)

    return pl.pallas_call(
        body,
        grid=(N // BM,),
        in_specs=[pl.BlockSpec((BM, 8), lambda i: (i, 0)),
                  pl.BlockSpec((BM, 64), lambda i: (i, 0)),
                  pl.BlockSpec((BM, 1), lambda i: (i, 0)),
                  pl.BlockSpec((64, 8), lambda i: (0, 0))],
        out_specs=pl.BlockSpec((BM, 8), lambda i: (i, 0)),
        out_shape=jax.ShapeDtypeStruct((N, 8), _f32),
    )(posP, s2, cnt, Wp8)


def _pad_rows(w, rows):
    return jnp.concatenate(
        [w, jnp.zeros((rows - w.shape[0], w.shape[1]), _f32)], axis=0)


def _gather_idx(faces, F):
    flat = jnp.concatenate([faces[:, 0], faces[:, 1], faces[:, 2]])
    Mpad = -(-3 * F // 4096) * 4096
    return jnp.concatenate(
        [flat, jnp.zeros((Mpad - 3 * F,), jnp.int32)]), Mpad


def _scatter_idx(faces, F, N):
    Ft = F // 16
    Kt = (Ft + 127) // 128
    cols = []
    for j in range(3):
        cj = faces[:, j].reshape(16, Ft)
        cj = jnp.pad(cj, ((0, 0), (0, Kt * 128 - Ft)), constant_values=N)
        cols.append(cj.reshape(16, Kt, 128))
    return jnp.stack(cols, axis=2)


def _scatter_idx_flat(faces, F, N):
    M = 3 * F
    Mt = M // 16
    Kt = (Mt + 127) // 128
    fl = jnp.concatenate([faces[:, 0], faces[:, 1], faces[:, 2]])
    fl = fl.reshape(16, Mt)
    fl = jnp.pad(fl, ((0, 0), (0, Kt * 128 - Mt)), constant_values=N)
    return fl.reshape(16, Kt, 1, 128)


def _build_faces(faces, N, F):
    a, b, c = faces[:, 0], faces[:, 1], faces[:, 2]
    i01 = N + jnp.arange(F, dtype=faces.dtype)
    i12 = i01 + F
    i20 = i12 + F
    c0 = jnp.stack([a, i01, i20], axis=1)
    c1 = jnp.stack([i01, b, i12], axis=1)
    c2 = jnp.stack([i20, i12, c], axis=1)
    c3 = jnp.stack([i01, i12, i20], axis=1)
    return jnp.stack([c0, c1, c2, c3], axis=1).reshape(-1, 3)


def _round(posP, faces, feat, cnt, Ws, Wn, b, Wp, Wf, bf, *, N, F, C,
           s1W, s1_sl, s2W, s2_sl, zblk, s1_pre=None, gidx_pre=None,
           sidx_pre=None, pg_pre=None):
    N_acc = -(-(N + 1) // 16) * 16
    if gidx_pre is not None:
        gidx, Mpad = gidx_pre
    else:
        gidx, Mpad = _gather_idx(faces, F)
    sidx = sidx_pre if sidx_pre is not None else _scatter_idx(faces, F, N)
    b2 = b.reshape(1, 64)
    A = Ws + (2.0 / 3.0) * Wn
    S = Ws + Wn
    Wn3 = Wn * (1.0 / 3.0)
    Wp8 = jnp.concatenate([Wp, jnp.zeros((64, 5), _f32)], axis=1)

    if s1_pre is not None:
        s1 = s1_pre
    else:
        s1 = _scatter3(feat, sidx, zblk, M=F, N_acc=N_acc, W=s1W,
                       n_slices=s1_sl, n_scat=3)
    nmW3 = _t1_nm_matmul(s1, cnt, Wn3, N, C)
    gnm = _gather(nmW3, gidx, 64, Mpad)
    pg = pg_pre if pg_pre is not None else _gather(posP, gidx, 8, Mpad)
    outs = _t2_children(feat, gnm, pg, A, S, b2, Wp8, F, C, Wf,
                        bf.reshape(1, 64) if bf is not None else None)
    if Wf is not None:
        g012, mp, nf = outs
        feat_next = nf
    else:
        g012, mp = outs
        feat_next = None
    sidx_f = _scatter_idx_flat(faces, F, N)
    s2 = _scatter3(g012.reshape(3 * F, 64), sidx_f, zblk, M=3 * F,
                   N_acc=N_acc, W=s2W, n_slices=s2_sl, n_scat=1)
    newposO = _t3_pos(posP, s2, cnt, Wp8, N)
    posP_next = jnp.concatenate(
        [newposO, mp[:, 0:8], mp[:, 8:16], mp[:, 16:24]], axis=0)
    faces_next = _build_faces(faces, N, F)
    cnt_next = jnp.concatenate([cnt, jnp.full((3 * F, 1), 3.0, _f32)], axis=0)
    return posP_next, faces_next, feat_next, cnt_next


def _scatter3(vals, sidx, zblk, *, M, N_acc, W, n_slices, n_scat):
    s_per = max(1, n_slices // 2)
    return _scatter_kernel(vals, sidx, zblk, M=M, N_acc=N_acc, W=W,
                           n_slices=n_slices, s_per=s_per, n_scat=n_scat)


def kernel(pos, faces, input_feature, Ws1, Wn1, b1, Wp1, Wf1, bf1,
           Ws2, Wn2, b2, Wp2, Wf2, bf2, Ws3, Wn3, b3, Wp3):
    N0, F0 = pos.shape[0], faces.shape[0]
    zblk = jnp.zeros((1024, 64), _f32)
    posP0 = jnp.concatenate([pos, jnp.zeros((N0, 5), _f32)], axis=1)

    gidx0, Mpad0 = _gather_idx(faces, F0)
    pg0 = _gather(posP0, gidx0, 8, Mpad0)
    ff = _t0_extract(pg0, input_feature, F0)

    Ws1p = _pad_rows(Ws1, 32)
    Wn1p = _pad_rows(Wn1, 32)

    N_acc0 = -(-(N0 + 1) // 16) * 16
    sidx0 = _scatter_idx(faces, F0, N0)
    s1r1 = _scatter3(ff, sidx0, zblk, M=F0, N_acc=N_acc0, W=16,
                     n_slices=2, n_scat=3)
    cnt1 = s1r1[:N0, 23:24]

    p1P, f1, feat1, cnt2 = _round(
        posP0, faces, ff, cnt1, Ws1p, Wn1p, b1, Wp1, Wf1, bf1,
        N=N0, F=F0, C=32, s1W=16, s1_sl=2, s2W=32, s2_sl=2, zblk=zblk,
        s1_pre=s1r1, gidx_pre=(gidx0, Mpad0), sidx_pre=sidx0, pg_pre=pg0,
    )

    N1, F1 = N0 + 3 * F0, 4 * F0
    p2P, f2, feat2, cnt3 = _round(
        p1P, f1, feat1, cnt2, Ws2, Wn2, b2, Wp2, Wf2, bf2,
        N=N1, F=F1, C=64, s1W=32, s1_sl=2, s2W=32, s2_sl=2, zblk=zblk,
    )

    N2, F2 = N1 + 3 * F1, 4 * F1
    p3P, f3, _, _ = _round(
        p2P, f2, feat2, cnt3, Ws3, Wn3, b3, Wp3, None, None,
        N=N2, F=F2, C=64, s1W=16, s1_sl=4, s2W=16, s2_sl=4, zblk=zblk,
    )

    return (p1P[:, 0:3], p2P[:, 0:3], p3P[:, 0:3], f1, f2, f3)


# speedup vs baseline: 9.1698x; 1.1735x over previous
import functools

import jax
import jax.numpy as jnp
from jax import lax
from jax.experimental import pallas as pl
from jax.experimental.pallas import tpu as pltpu
from jax.experimental.pallas import tpu_sc as plsc

BM = 400

_f32 = jnp.float32


def _mesh():
    return plsc.VectorSubcoreMesh(core_axis_name="c", subcore_axis_name="s")


_SC_PARAMS = pltpu.CompilerParams(use_tc_tiling_on_sc=False)


def _gather(table, idx, W, Mpad):
    Mt = Mpad // 32
    n_g = Mt // 128
    idx = idx.reshape(Mpad // 128, 128)

    @functools.partial(
        pl.kernel,
        out_type=jax.ShapeDtypeStruct((Mpad, W), _f32),
        mesh=_mesh(),
        compiler_params=_SC_PARAMS,
        scratch_types=[
            pltpu.VMEM((n_g, 128), jnp.int32),
            pltpu.VMEM((2, 128, W), _f32),
            pltpu.SemaphoreType.DMA,
            pltpu.SemaphoreType.DMA,
            pltpu.SemaphoreType.DMA,
            pltpu.SemaphoreType.DMA,
        ],
    )
    def k(table_hbm, idx_hbm, out_hbm, idx_v, rows_v, g0, g1, s0, s1):
        c = lax.axis_index("c")
        s = lax.axis_index("s")
        wid = s * 2 + c
        base = wid * Mt
        gsem = (g0, g1)
        ssem = (s0, s1)

        pltpu.sync_copy(idx_hbm.at[pl.ds(wid * n_g, n_g)], idx_v)

        def g_issue(i, b):
            pltpu.async_copy(table_hbm.at[idx_v.at[i]], rows_v.at[b],
                             gsem[b])

        def g_wait(i, b):
            pltpu.make_async_copy(table_hbm.at[idx_v.at[i]], rows_v.at[b],
                                  gsem[b]).wait()

        def s_issue(i, b):
            pltpu.async_copy(rows_v.at[b],
                             out_hbm.at[pl.ds(base + i * 128, 128)], ssem[b])

        def s_wait(b):
            pltpu.make_async_copy(rows_v.at[b],
                                  out_hbm.at[pl.ds(0, 128)], ssem[b]).wait()

        g_issue(0, 0)

        def body(p, _):
            i0 = 2 * p

            @pl.when(p > 0)
            def _():
                s_wait(1)

            g_issue(i0 + 1, 1)
            g_wait(i0, 0)
            s_issue(i0, 0)

            @pl.when(p < n_g // 2 - 1)
            def _():
                s_wait(0)
                g_issue(i0 + 2, 0)

            g_wait(i0 + 1, 1)
            s_issue(i0 + 1, 1)
            return _

        lax.fori_loop(0, n_g // 2, body, None)
        s_wait(0)
        s_wait(1)

    return k(table, idx)


def _scatter_kernel(vals, idx3, zblk, *, M, N_acc, W, n_slices, s_per,
                    n_scat):
    Mt = M // 16
    Kt = (Mt + 127) // 128
    n_full = Mt // 128
    rem = Mt - n_full * 128
    Nz = N_acc // 16
    n_zf = Nz // 1024
    zrem = Nz - n_zf * 1024
    Ctot = n_slices * W

    @functools.partial(
        pl.kernel,
        out_type=jax.ShapeDtypeStruct((N_acc, Ctot), _f32),
        mesh=_mesh(),
        compiler_params=_SC_PARAMS,
        scratch_types=[
            pltpu.VMEM((2, n_scat, 128), jnp.int32),
            pltpu.VMEM((2, 128, W), _f32),
            pltpu.VMEM_SHARED((N_acc, W), _f32),
            pltpu.SemaphoreType.DMA,
            pltpu.SemaphoreType.DMA,
            pltpu.SemaphoreType.DMA,
            pltpu.SemaphoreType.DMA,
        ],
    )
    def k(vals_hbm, idx_hbm, z_hbm, out_hbm, idx_v, rows_v, acc,
          sm0, sm1, lm0, lm1):
        c = lax.axis_index("c")
        t = lax.axis_index("s")
        base_n = t * Nz
        f_base = t * Mt
        sems = (sm0, sm1)
        lsem = (lm0, lm1)

        def drain(b):
            for j in range(n_scat):
                pltpu.make_async_copy(rows_v.at[b],
                                      acc.at[idx_v.at[b, j]],
                                      sems[b]).wait()

        for slice_id in range(n_slices):
            owner = slice_id // s_per

            @pl.when(c == owner)
            def _():
                c0 = slice_id * W
                for z in range(n_zf):
                    pltpu.sync_copy(z_hbm.at[pl.ds(0, 1024), pl.ds(0, W)],
                                    acc.at[pl.ds(base_n + z * 1024, 1024)])
                if zrem:
                    pltpu.sync_copy(z_hbm.at[pl.ds(0, zrem), pl.ds(0, W)],
                                    acc.at[pl.ds(base_n + n_zf * 1024, zrem)])
                plsc.subcore_barrier()

                def loads(i, b):
                    pltpu.async_copy(idx_hbm.at[t, i], idx_v.at[b], lsem[b])
                    pltpu.async_copy(
                        vals_hbm.at[pl.ds(f_base + i * 128, 128),
                                    pl.ds(c0, W)], rows_v.at[b], lsem[b])

                def wait_loads(b):
                    pltpu.make_async_copy(idx_hbm.at[t, 0], idx_v.at[b],
                                          lsem[b]).wait()
                    pltpu.make_async_copy(
                        vals_hbm.at[pl.ds(f_base, 128), pl.ds(c0, W)],
                        rows_v.at[b], lsem[b]).wait()

                def scats(b):
                    for j in range(n_scat):
                        pltpu.async_copy(rows_v.at[b],
                                         acc.at[idx_v.at[b, j]],
                                         sems[b], add=True)

                loads(0, 0)

                def pbody(p, _):
                    i0 = 2 * p

                    @pl.when(p > 0)
                    def _():
                        drain(1)

                    loads(i0 + 1, 1)
                    wait_loads(0)
                    scats(0)

                    @pl.when(i0 + 2 < n_full)
                    def _():
                        drain(0)
                        loads(i0 + 2, 0)

                    wait_loads(1)
                    scats(1)
                    return _

                n_pairs = n_full // 2
                lax.fori_loop(0, n_pairs, pbody, None)
                if n_full % 2:
                    wait_loads(0)
                    scats(0)
                if rem:
                    b = n_full % 2
                    drain(b)
                    pltpu.sync_copy(idx_hbm.at[t, n_full], idx_v.at[b])
                    pltpu.sync_copy(
                        vals_hbm.at[pl.ds(f_base + n_full * 128, rem),
                                    pl.ds(c0, W)],
                        rows_v.at[b, pl.ds(0, rem)])
                    scats(b)
                drain(0)
                drain(1)
                plsc.subcore_barrier()
                pltpu.sync_copy(acc.at[pl.ds(base_n, Nz)],
                                out_hbm.at[pl.ds(base_n, Nz), pl.ds(c0, W)])

    return k(vals, idx3, zblk)


def _t0_extract(pg, infeat, F):
    def body(pa, pb, pc, x, o):
        v0, v1, v2 = pa[:, 0:3], pb[:, 0:3], pc[:, 0:3]
        center = (v0 + v1 + v2) * (1.0 / 3.0)
        e1 = v1 - v0
        e2 = v2 - v0
        cr = jnp.stack([
            e1[:, 1] * e2[:, 2] - e1[:, 2] * e2[:, 1],
            e1[:, 2] * e2[:, 0] - e1[:, 0] * e2[:, 2],
            e1[:, 0] * e2[:, 1] - e1[:, 1] * e2[:, 0],
        ], axis=1)
        area = 0.5 * jnp.sqrt(jnp.sum(cr * cr, axis=1, keepdims=True) + 1e-12)
        normal = cr / (2.0 * area + 1e-8)
        ones = jnp.ones((pa.shape[0], 1), _f32)
        zer = jnp.zeros((pa.shape[0], 8), _f32)
        o[...] = jnp.concatenate([center, normal, area, x[...], ones, zer],
                                 axis=1)

    nb = F // BM
    sp8 = lambda j: pl.BlockSpec((BM, 8), lambda i, j=j: (i + j * nb, 0))
    return pl.pallas_call(
        body,
        grid=(nb,),
        in_specs=[sp8(0), sp8(1), sp8(2),
                  pl.BlockSpec((BM, 16), lambda i: (i, 0))],
        out_specs=pl.BlockSpec((BM, 32), lambda i: (i, 0)),
        out_shape=jax.ShapeDtypeStruct((F, 32), _f32),
    )(pg, pg, pg, infeat)


def _t1_nm_matmul(s1, cnt, W, N, C):
    def body(s_ref, c_ref, w_ref, o):
        inv = 1.0 / jnp.maximum(c_ref[...], 1.0)
        o[...] = jnp.dot(s_ref[...] * inv, w_ref[...],
                         preferred_element_type=_f32)

    return pl.pallas_call(
        body,
        grid=(N // BM,),
        in_specs=[pl.BlockSpec((BM, C), lambda i: (i, 0)),
                  pl.BlockSpec((BM, 1), lambda i: (i, 0)),
                  pl.BlockSpec((C, 64), lambda i: (0, 0))],
        out_specs=pl.BlockSpec((BM, 64), lambda i: (i, 0)),
        out_shape=jax.ShapeDtypeStruct((N, 64), _f32),
    )(s1, cnt, W)


def _t2_children(feat, gnm, pg, A, S, b, Wp8, F, C, Wf=None, bf=None):
    with_nf = Wf is not None

    def body(*refs):
        if with_nf:
            (f_ref, g0r, g1r, g2r, pa, pb, pc, a_ref, s_ref, b_ref, wp_ref,
             wf_ref, bf_ref, g3d_o, mp_o, nf_o) = refs
        else:
            (f_ref, g0r, g1r, g2r, pa, pb, pc, a_ref, s_ref, b_ref,
             wp_ref, g3d_o, mp_o) = refs
        ft = f_ref[...]
        bb = b_ref[...]
        P = jnp.dot(ft, a_ref[...], preferred_element_type=_f32) + bb
        Q = jnp.dot(ft, s_ref[...], preferred_element_type=_f32) + bb
        g0 = jnp.maximum(P + g0r[...], 0.0)
        g1 = jnp.maximum(P + g1r[...], 0.0)
        g2 = jnp.maximum(P + g2r[...], 0.0)
        g3 = jnp.maximum(Q, 0.0)
        g3d_o[...] = jnp.stack([g0, g1, g2], axis=0)
        third = 1.0 / 3.0
        wp = wp_ref[...]
        m01 = jnp.dot((g0 + g1 + g3) * third, wp, preferred_element_type=_f32)
        m12 = jnp.dot((g1 + g2 + g3) * third, wp, preferred_element_type=_f32)
        m20 = jnp.dot((g0 + g2 + g3) * third, wp, preferred_element_type=_f32)
        p01 = 0.5 * (pa[...] + pb[...]) + m01
        p12 = 0.5 * (pb[...] + pc[...]) + m12
        p20 = 0.5 * (pc[...] + pa[...]) + m20
        mp_o[...] = jnp.stack([p01, p12, p20], axis=0)
        if with_nf:
            wf = wf_ref[...]
            bf_ = bf_ref[...]
            nf_o[...] = jnp.stack(
                [jnp.dot(g, wf, preferred_element_type=_f32) + bf_
                 for g in (g0, g1, g2, g3)],
                axis=1).reshape(4 * g0.shape[0], 64)

    nb = F // BM
    sp64 = lambda j: pl.BlockSpec((BM, 64), lambda i, j=j: (i + j * nb, 0))
    sp8 = lambda j: pl.BlockSpec((BM, 8), lambda i, j=j: (i + j * nb, 0))
    full = lambda r, c: pl.BlockSpec((r, c), lambda i: (0, 0))
    in_specs = [pl.BlockSpec((BM, C), lambda i: (i, 0)),
                sp64(0), sp64(1), sp64(2), sp8(0), sp8(1), sp8(2),
                full(C, 64), full(C, 64), full(1, 64), full(64, 8)]
    args = [feat, gnm, gnm, gnm, pg, pg, pg, A, S, b, Wp8]
    out_specs = [pl.BlockSpec((3, BM, 64), lambda i: (0, i, 0)),
                 pl.BlockSpec((3, BM, 8), lambda i: (0, i, 0))]
    out_shape = [jax.ShapeDtypeStruct((3, F, 64), _f32),
                 jax.ShapeDtypeStruct((3, F, 8), _f32)]
    if with_nf:
        in_specs += [full(64, 64), full(1, 64)]
        args += [Wf, bf]
        out_specs.append(pl.BlockSpec((4 * BM, 64), lambda i: (i, 0)))
        out_shape.append(jax.ShapeDtypeStruct((4 * F, 64), _f32))
    return pl.pallas_call(
        body, grid=(nb,), in_specs=in_specs, out_specs=out_specs,
        out_shape=out_shape)(*args)


def _t3_pos(posP, s2, cnt, Wp8, N):
    def body(p_ref, s_ref, c_ref, w_ref, o):
        inv = 1.0 / jnp.maximum(c_ref[...], 1.0)
        o[...] = p_ref[...] + jnp.dot(s_ref[...] * inv, w_ref[...],
                                      preferred_element_type=_f32)

    return pl.pallas_call(
        body,
        grid=(N // BM,),
        in_specs=[pl.BlockSpec((BM, 8), lambda i: (i, 0)),
                  pl.BlockSpec((BM, 64), lambda i: (i, 0)),
                  pl.BlockSpec((BM, 1), lambda i: (i, 0)),
                  pl.BlockSpec((64, 8), lambda i: (0, 0))],
        out_specs=pl.BlockSpec((BM, 8), lambda i: (i, 0)),
        out_shape=jax.ShapeDtypeStruct((N, 8), _f32),
    )(posP, s2, cnt, Wp8)


def _pad_rows(w, rows):
    return jnp.concatenate(
        [w, jnp.zeros((rows - w.shape[0], w.shape[1]), _f32)], axis=0)


def _gather_idx(faces, F):
    flat = jnp.concatenate([faces[:, 0], faces[:, 1], faces[:, 2]])
    Mpad = -(-3 * F // 4096) * 4096
    return jnp.concatenate(
        [flat, jnp.zeros((Mpad - 3 * F,), jnp.int32)]), Mpad


def _scatter_idx(faces, F, N):
    Ft = F // 16
    Kt = (Ft + 127) // 128
    cols = []
    for j in range(3):
        cj = faces[:, j].reshape(16, Ft)
        cj = jnp.pad(cj, ((0, 0), (0, Kt * 128 - Ft)), constant_values=N)
        cols.append(cj.reshape(16, Kt, 128))
    return jnp.stack(cols, axis=2)


def _scatter_idx_flat(faces, F, N):
    M = 3 * F
    Mt = M // 16
    Kt = (Mt + 127) // 128
    fl = jnp.concatenate([faces[:, 0], faces[:, 1], faces[:, 2]])
    fl = fl.reshape(16, Mt)
    fl = jnp.pad(fl, ((0, 0), (0, Kt * 128 - Mt)), constant_values=N)
    return fl.reshape(16, Kt, 1, 128)


def _build_faces(faces, N, F):
    a, b, c = faces[:, 0], faces[:, 1], faces[:, 2]
    i01 = N + jnp.arange(F, dtype=faces.dtype)
    i12 = i01 + F
    i20 = i12 + F
    c0 = jnp.stack([a, i01, i20], axis=1)
    c1 = jnp.stack([i01, b, i12], axis=1)
    c2 = jnp.stack([i20, i12, c], axis=1)
    c3 = jnp.stack([i01, i12, i20], axis=1)
    return jnp.stack([c0, c1, c2, c3], axis=1).reshape(-1, 3)


def _round(posP, faces, feat, cnt, Ws, Wn, b, Wp, Wf, bf, *, N, F, C,
           s1W, s1_sl, s2W, s2_sl, zblk, s1_pre=None, gidx_pre=None,
           sidx_pre=None, pg_pre=None):
    N_acc = -(-(N + 1) // 16) * 16
    if gidx_pre is not None:
        gidx, Mpad = gidx_pre
    else:
        gidx, Mpad = _gather_idx(faces, F)
    sidx = sidx_pre if sidx_pre is not None else _scatter_idx(faces, F, N)
    b2 = b.reshape(1, 64)
    A = Ws + (2.0 / 3.0) * Wn
    S = Ws + Wn
    Wn3 = Wn * (1.0 / 3.0)
    Wp8 = jnp.concatenate([Wp, jnp.zeros((64, 5), _f32)], axis=1)

    if s1_pre is not None:
        s1 = s1_pre
    else:
        s1 = _scatter3(feat, sidx, zblk, M=F, N_acc=N_acc, W=s1W,
                       n_slices=s1_sl, n_scat=3)
    nmW3 = _t1_nm_matmul(s1, cnt, Wn3, N, C)
    gnm = _gather(nmW3, gidx, 64, Mpad)
    pg = pg_pre if pg_pre is not None else _gather(posP, gidx, 8, Mpad)
    outs = _t2_children(feat, gnm, pg, A, S, b2, Wp8, F, C, Wf,
                        bf.reshape(1, 64) if bf is not None else None)
    if Wf is not None:
        g012, mp, nf = outs
        feat_next = nf
    else:
        g012, mp = outs
        feat_next = None
    sidx_f = _scatter_idx_flat(faces, F, N)
    s2 = _scatter3(g012.reshape(3 * F, 64), sidx_f, zblk, M=3 * F,
                   N_acc=N_acc, W=s2W, n_slices=s2_sl, n_scat=1)
    newposO = _t3_pos(posP, s2, cnt, Wp8, N)
    posP_next = jnp.concatenate([newposO, mp.reshape(3 * F, 8)], axis=0)
    faces_next = _build_faces(faces, N, F)
    cnt_next = jnp.concatenate([cnt, jnp.full((3 * F, 1), 3.0, _f32)], axis=0)
    return posP_next, faces_next, feat_next, cnt_next


def _scatter3(vals, sidx, zblk, *, M, N_acc, W, n_slices, n_scat):
    s_per = max(1, n_slices // 2)
    return _scatter_kernel(vals, sidx, zblk, M=M, N_acc=N_acc, W=W,
                           n_slices=n_slices, s_per=s_per, n_scat=n_scat)


def kernel(pos, faces, input_feature, Ws1, Wn1, b1, Wp1, Wf1, bf1,
           Ws2, Wn2, b2, Wp2, Wf2, bf2, Ws3, Wn3, b3, Wp3):
    N0, F0 = pos.shape[0], faces.shape[0]
    zblk = jnp.zeros((1024, 64), _f32)
    posP0 = jnp.concatenate([pos, jnp.zeros((N0, 5), _f32)], axis=1)

    gidx0, Mpad0 = _gather_idx(faces, F0)
    pg0 = _gather(posP0, gidx0, 8, Mpad0)
    ff = _t0_extract(pg0, input_feature, F0)

    Ws1p = _pad_rows(Ws1, 32)
    Wn1p = _pad_rows(Wn1, 32)

    N_acc0 = -(-(N0 + 1) // 16) * 16
    sidx0 = _scatter_idx(faces, F0, N0)
    s1r1 = _scatter3(ff, sidx0, zblk, M=F0, N_acc=N_acc0, W=16,
                     n_slices=2, n_scat=3)
    cnt1 = s1r1[:N0, 23:24]

    p1P, f1, feat1, cnt2 = _round(
        posP0, faces, ff, cnt1, Ws1p, Wn1p, b1, Wp1, Wf1, bf1,
        N=N0, F=F0, C=32, s1W=16, s1_sl=2, s2W=32, s2_sl=2, zblk=zblk,
        s1_pre=s1r1, gidx_pre=(gidx0, Mpad0), sidx_pre=sidx0, pg_pre=pg0,
    )

    N1, F1 = N0 + 3 * F0, 4 * F0
    p2P, f2, feat2, cnt3 = _round(
        p1P, f1, feat1, cnt2, Ws2, Wn2, b2, Wp2, Wf2, bf2,
        N=N1, F=F1, C=64, s1W=32, s1_sl=2, s2W=32, s2_sl=2, zblk=zblk,
    )

    N2, F2 = N1 + 3 * F1, 4 * F1
    p3P, f3, _, _ = _round(
        p2P, f2, feat2, cnt3, Ws3, Wn3, b3, Wp3, None, None,
        N=N2, F=F2, C=64, s1W=16, s1_sl=4, s2W=16, s2_sl=4, zblk=zblk,
    )

    return (p1P[:, 0:3], p2P[:, 0:3], p3P[:, 0:3], f1, f2, f3)
